# baseline (speedup 1.0000x reference)
import functools

import jax
import jax.numpy as jnp
from jax import lax
from jax.experimental import pallas as pl
from jax.experimental.pallas import tpu as pltpu
from jax.experimental.pallas import tpu_sc as plsc

N = 10000
E = 160000
F = 128
G = 100
NPER = 100
NB = 1000
GRID = N // NB

NCORES = 2
AGG_PARTS = 1
CHUNK = 128
CHUNKS_PW = 80
HALF = CHUNKS_PW // 2
NCHUNKS_ALLOC = 16 * CHUNKS_PW
EPAD = NCHUNKS_ALLOC * CHUNK
NPAD = 10112
ROWS_PT = NPAD // 16


def _embed_body(x_ref, w0t_ref, conv0_ref, h_ref, m_ref):
    h = jax.nn.sigmoid(jnp.dot(x_ref[...], w0t_ref[...],
                               preferred_element_type=jnp.float32))
    h_ref[...] = h
    m_ref[...] = jnp.dot(h, conv0_ref[...], preferred_element_type=jnp.float32)


def _gru_body(*refs, with_conv):
    aggs = refs[:AGG_PARTS]
    (h_ref, wih_ref, whh_ref, bih_ref, bhh_ref, conv_ref, h_out_ref,
     *rest) = refs[AGG_PARTS:]
    m_ref = rest[0] if with_conv else None
    agg = aggs[0][...]
    for a in aggs[1:]:
        agg = agg + a[...]
    h = h_ref[...]
    gi = jnp.dot(agg, wih_ref[...], preferred_element_type=jnp.float32) + bih_ref[...]
    gh = jnp.dot(h, whh_ref[...], preferred_element_type=jnp.float32) + bhh_ref[...]
    r = jax.nn.sigmoid(gi[:, :F] + gh[:, :F])
    z = jax.nn.sigmoid(gi[:, F:2 * F] + gh[:, F:2 * F])
    n = jnp.tanh(gi[:, 2 * F:] + r * gh[:, 2 * F:])
    hn = (1.0 - z) * n + z * h
    h_out_ref[...] = hn
    if with_conv:
        m_ref[...] = jnp.dot(hn, conv_ref[...], preferred_element_type=jnp.float32)


def _softplus(x):
    return jnp.maximum(x, 0.0) + jnp.log1p(jnp.exp(-jnp.abs(x)))


def _head_sample_body(h3_ref, w1_ref, b1_ref, w2_ref, b2_ref, noise_ref, out_ref):
    h3 = jnp.maximum(h3_ref[...], 0.0)
    w1 = w1_ref[...].reshape(1, 1, F)
    w2 = w2_ref[...].reshape(1, 1, F)
    mu = jnp.sum(h3 * w1, axis=2) + b1_ref[0, 0]
    sigma = _softplus(jnp.sum(h3 * w2, axis=2) + b2_ref[0, 0])

    col = lax.broadcasted_iota(jnp.int32, (G, NPER), 1)
    row_t = lax.broadcasted_iota(jnp.int32, (NPER, NPER), 0)
    col_t = lax.broadcasted_iota(jnp.int32, (NPER, NPER), 1)
    main = col < (NPER - 1)
    lastc = col == (NPER - 1)

    s = jnp.where(main, sigma, 0.0)
    mus = jnp.where(main, mu, 0.0)
    sn = jnp.sum(jnp.where(lastc, sigma, 0.0), axis=1, keepdims=True)
    mun = jnp.sum(jnp.where(lastc, mu, 0.0), axis=1, keepdims=True)

    d = s + 1e-6
    sum_s = jnp.sum(s, axis=1, keepdims=True)
    sum_mu = jnp.sum(mus, axis=1, keepdims=True)
    tot = sn + sum_s
    c0 = 1.0 / tot
    c = -mun / sn
    rmean = c * s + mus - c0 * (c * sum_s + sum_mu) * s

    t_rev = (row_t >= col_t).astype(jnp.float32)
    t_ex = (row_t < col_t).astype(jnp.float32)
    rev = jnp.dot(s, t_rev, preferred_element_type=jnp.float32)
    cex = jnp.dot(s / d, t_ex, preferred_element_type=jnp.float32)
    t = 1.0 / (-(sn + rev + 1e-6 * cex))
    ell = jnp.sqrt(d + t * s * s)
    w = t * s / ell

    noise = noise_ref[...]
    wn = w * noise
    cum_wn = jnp.dot(wn, t_ex, preferred_element_type=jnp.float32)
    xr = rmean + ell * noise + s * cum_wn
    xr = jnp.where(main, xr, 0.0)
    last = -jnp.sum(xr, axis=1, keepdims=True)
    out_ref[...] = jnp.where(lastc, jnp.broadcast_to(last, (G, NPER)), xr)


def _sc_segsum_body(m_hbm, src_hbm, dst_hbm, zeros_hbm, out_hbm,
                    srcv, dstv, rowsv, aggsh, sem):
    c = lax.axis_index("c")
    s = lax.axis_index("s")

    @pl.when(c == 0)
    def _core0_body():
        _sc_core0_work(m_hbm, src_hbm, dst_hbm, zeros_hbm, out_hbm,
                       srcv, dstv, rowsv, aggsh, sem, s)


def _sc_core0_work(m_hbm, src_hbm, dst_hbm, zeros_hbm, out_hbm,
                   srcv, dstv, rowsv, aggsh, sem, s):
    pltpu.sync_copy(zeros_hbm.at[pl.ds(s * ROWS_PT, ROWS_PT)],
                    aggsh.at[pl.ds(s * ROWS_PT, ROWS_PT)])
    plsc.subcore_barrier()

    rows0, rows1 = rowsv.at[0], rowsv.at[1]
    sem0, sem1 = sem.at[0], sem.at[1]

    def _gather(j, buf, s_):
        pltpu.async_copy(m_hbm.at[srcv.at[j]], buf, s_)

    def _drain(buf, s_):
        pltpu.make_async_copy(m_hbm.at[srcv.at[0]], buf, s_).wait()

    def _scatter(j, buf):
        pltpu.sync_copy(buf, aggsh.at[dstv.at[j]], add=True)

    @pl.loop(0, CHUNKS_PW // HALF)
    def _half(half):
        base = s * CHUNKS_PW + half * HALF
        pltpu.sync_copy(src_hbm.at[pl.ds(base, HALF)], srcv)
        pltpu.sync_copy(dst_hbm.at[pl.ds(base, HALF)], dstv)

        _gather(0, rows0, sem0)

        @pl.loop(0, HALF // 2 - 1)
        def _pair(k):
            j = 2 * k
            _gather(j + 1, rows1, sem1)
            _drain(rows0, sem0)
            _scatter(j, rows0)
            _gather(j + 2, rows0, sem0)
            _drain(rows1, sem1)
            _scatter(j + 1, rows1)

        _gather(HALF - 1, rows1, sem1)
        _drain(rows0, sem0)
        _scatter(HALF - 2, rows0)
        _drain(rows1, sem1)
        _scatter(HALF - 1, rows1)

    plsc.subcore_barrier()
    pltpu.sync_copy(aggsh.at[pl.ds(s * ROWS_PT, ROWS_PT)],
                    out_hbm.at[0, pl.ds(s * ROWS_PT, ROWS_PT)])


@functools.cache
def _get_sc_segsum():
    return pl.kernel(
        _sc_segsum_body,
        out_type=jax.ShapeDtypeStruct((AGG_PARTS, NPAD, F), jnp.float32),
        mesh=plsc.VectorSubcoreMesh(core_axis_name="c", subcore_axis_name="s",
                                    num_cores=NCORES),
        scratch_types=[
            pltpu.VMEM((HALF, CHUNK), jnp.int32),
            pltpu.VMEM((HALF, CHUNK), jnp.int32),
            pltpu.VMEM((2, CHUNK, F), jnp.float32),
            pltpu.VMEM_SHARED((NPAD, F), jnp.float32),
            pltpu.SemaphoreType.DMA((2,)),
        ],
    )


def _embed_call(x, w0t, conv0):
    return pl.pallas_call(
        _embed_body,
        grid=(GRID,),
        in_specs=[
            pl.BlockSpec((NB, F), lambda i: (i, 0)),
            pl.BlockSpec((F, F), lambda i: (0, 0)),
            pl.BlockSpec((F, F), lambda i: (0, 0)),
        ],
        out_specs=[
            pl.BlockSpec((NB, F), lambda i: (i, 0)),
            pl.BlockSpec((NB, F), lambda i: (i, 0)),
        ],
        out_shape=[
            jax.ShapeDtypeStruct((N, F), jnp.float32),
            jax.ShapeDtypeStruct((N, F), jnp.float32),
        ],
    )(x, w0t, conv0)


def _gru_call(aggs, h, wih, whh, bih, bhh, conv, with_conv):
    full = lambda i: (0, 0)
    blk = lambda i: (i, 0)
    out_shape = [jax.ShapeDtypeStruct((N, F), jnp.float32)]
    out_specs = [pl.BlockSpec((NB, F), blk)]
    if with_conv:
        out_shape.append(jax.ShapeDtypeStruct((N, F), jnp.float32))
        out_specs.append(pl.BlockSpec((NB, F), blk))
    return pl.pallas_call(
        functools.partial(_gru_body, with_conv=with_conv),
        grid=(GRID,),
        in_specs=[pl.BlockSpec((NB, F), blk)] * AGG_PARTS + [
            pl.BlockSpec((NB, F), blk),
            pl.BlockSpec((F, 3 * F), full),
            pl.BlockSpec((F, 3 * F), full),
            pl.BlockSpec((1, 3 * F), full),
            pl.BlockSpec((1, 3 * F), full),
            pl.BlockSpec((F, F), full),
        ],
        out_specs=out_specs,
        out_shape=out_shape,
    )(*aggs, h, wih, whh, bih, bhh, conv)


def _head_sample_call(h3, w1, b1, w2, b2, noise):
    return pl.pallas_call(
        _head_sample_body,
        out_shape=jax.ShapeDtypeStruct((G, NPER), jnp.float32),
    )(h3, w1, b1, w2, b2, noise)


def kernel(x, edge_index, batch, num_graphs, W0, conv_weight, gru_w_ih,
           gru_w_hh, gru_b_ih, gru_b_hh, w1, b1, w2, b2):
    w0t = W0.T
    wih = gru_w_ih.T
    whh = gru_w_hh.T
    bih = gru_b_ih.reshape(1, 3 * F)
    bhh = gru_b_hh.reshape(1, 3 * F)

    src = edge_index[0]
    dst = edge_index[1]
    pad = NCHUNKS_ALLOC * CHUNK - E
    src_p = jnp.concatenate([src, jnp.zeros((pad,), jnp.int32)])
    dst_p = jnp.concatenate([dst, jnp.full((pad,), N, jnp.int32)])
    src2 = src_p.reshape(NCHUNKS_ALLOC, CHUNK)
    dst2 = dst_p.reshape(NCHUNKS_ALLOC, CHUNK)
    zeros = jnp.zeros((NPAD, F), jnp.float32)

    h, m = _embed_call(x, w0t, conv_weight[0])
    sc_segsum = _get_sc_segsum()
    for i in range(3):
        parts = sc_segsum(m, src2, dst2, zeros)
        aggs = [parts[k, :N] for k in range(AGG_PARTS)]
        with_conv = i < 2
        conv_next = conv_weight[i + 1] if with_conv else conv_weight[0]
        res = _gru_call(aggs, h, wih, whh, bih, bhh, conv_next, with_conv)
        if with_conv:
            h, m = res
        else:
            h = res[0] if isinstance(res, (list, tuple)) else res

    h3 = h.reshape(G, NPER, F)
    noise = jax.random.normal(jax.random.key(42), (G, NPER - 1), jnp.float32)
    noise_p = jnp.pad(noise, ((0, 0), (0, 1)))
    pred = _head_sample_call(h3, w1.reshape(1, F), b1.reshape(1, 1),
                             w2.reshape(1, F), b2.reshape(1, 1), noise_p)
    return pred.reshape(-1)


# speedup vs baseline: 2.1989x; 2.1989x over previous
import functools

import jax
import jax.numpy as jnp
from jax import lax
from jax.experimental import pallas as pl
from jax.experimental.pallas import tpu as pltpu
from jax.experimental.pallas import tpu_sc as plsc

N = 10000
E = 160000
F = 128
G = 100
NPER = 100
NB = 1000
GRID = N // NB

NCORES = 2
AGG_PARTS = 1
CHUNK = 128
CHUNKS_PW = 80
HALF = CHUNKS_PW // 2
NCHUNKS_ALLOC = 16 * CHUNKS_PW
EPAD = NCHUNKS_ALLOC * CHUNK
NPAD = 10112
ROWS_PT = NPAD // 16


def _embed_body(x_ref, w0t_ref, conv0_ref, h_ref, m_ref):
    h = jax.nn.sigmoid(jnp.dot(x_ref[...], w0t_ref[...],
                               preferred_element_type=jnp.float32))
    h_ref[...] = h
    m_ref[...] = jnp.dot(h, conv0_ref[...], preferred_element_type=jnp.float32)


def _gru_body(*refs, with_conv):
    aggs = refs[:AGG_PARTS]
    (h_ref, wih_ref, whh_ref, bih_ref, bhh_ref, conv_ref, h_out_ref,
     *rest) = refs[AGG_PARTS:]
    m_ref = rest[0] if with_conv else None
    agg = aggs[0][...]
    for a in aggs[1:]:
        agg = agg + a[...]
    h = h_ref[...]
    gi = jnp.dot(agg, wih_ref[...], preferred_element_type=jnp.float32) + bih_ref[...]
    gh = jnp.dot(h, whh_ref[...], preferred_element_type=jnp.float32) + bhh_ref[...]
    r = jax.nn.sigmoid(gi[:, :F] + gh[:, :F])
    z = jax.nn.sigmoid(gi[:, F:2 * F] + gh[:, F:2 * F])
    n = jnp.tanh(gi[:, 2 * F:] + r * gh[:, 2 * F:])
    hn = (1.0 - z) * n + z * h
    h_out_ref[...] = hn
    if with_conv:
        m_ref[...] = jnp.dot(hn, conv_ref[...], preferred_element_type=jnp.float32)


def _softplus(x):
    return jnp.maximum(x, 0.0) + jnp.log1p(jnp.exp(-jnp.abs(x)))


def _head_sample_body(h3_ref, w1_ref, b1_ref, w2_ref, b2_ref, noise_ref, out_ref):
    h3 = jnp.maximum(h3_ref[...], 0.0)
    w1 = w1_ref[...].reshape(1, 1, F)
    w2 = w2_ref[...].reshape(1, 1, F)
    mu = jnp.sum(h3 * w1, axis=2) + b1_ref[0, 0]
    sigma = _softplus(jnp.sum(h3 * w2, axis=2) + b2_ref[0, 0])

    col = lax.broadcasted_iota(jnp.int32, (G, NPER), 1)
    row_t = lax.broadcasted_iota(jnp.int32, (NPER, NPER), 0)
    col_t = lax.broadcasted_iota(jnp.int32, (NPER, NPER), 1)
    main = col < (NPER - 1)
    lastc = col == (NPER - 1)

    s = jnp.where(main, sigma, 0.0)
    mus = jnp.where(main, mu, 0.0)
    sn = jnp.sum(jnp.where(lastc, sigma, 0.0), axis=1, keepdims=True)
    mun = jnp.sum(jnp.where(lastc, mu, 0.0), axis=1, keepdims=True)

    d = s + 1e-6
    sum_s = jnp.sum(s, axis=1, keepdims=True)
    sum_mu = jnp.sum(mus, axis=1, keepdims=True)
    tot = sn + sum_s
    c0 = 1.0 / tot
    c = -mun / sn
    rmean = c * s + mus - c0 * (c * sum_s + sum_mu) * s

    t_rev = (row_t >= col_t).astype(jnp.float32)
    t_ex = (row_t < col_t).astype(jnp.float32)
    rev = jnp.dot(s, t_rev, preferred_element_type=jnp.float32)
    cex = jnp.dot(s / d, t_ex, preferred_element_type=jnp.float32)
    t = 1.0 / (-(sn + rev + 1e-6 * cex))
    ell = jnp.sqrt(d + t * s * s)
    w = t * s / ell

    noise = noise_ref[...]
    wn = w * noise
    cum_wn = jnp.dot(wn, t_ex, preferred_element_type=jnp.float32)
    xr = rmean + ell * noise + s * cum_wn
    xr = jnp.where(main, xr, 0.0)
    last = -jnp.sum(xr, axis=1, keepdims=True)
    out_ref[...] = jnp.where(lastc, jnp.broadcast_to(last, (G, NPER)), xr)


def _sc_segsum_body(m_hbm, src_hbm, dst_hbm, zeros_hbm, out_hbm,
                    srcv, dstv, rowsv, aggsh, sem):
    c = lax.axis_index("c")
    s = lax.axis_index("s")

    @pl.when(c == 0)
    def _core0_body():
        _sc_core0_work(m_hbm, src_hbm, dst_hbm, zeros_hbm, out_hbm,
                       srcv, dstv, rowsv, aggsh, sem, s)


def _sc_core0_work(m_hbm, src_hbm, dst_hbm, zeros_hbm, out_hbm,
                   srcv, dstv, rowsv, aggsh, sem, s):
    pltpu.sync_copy(zeros_hbm.at[pl.ds(s * ROWS_PT, ROWS_PT)],
                    aggsh.at[pl.ds(s * ROWS_PT, ROWS_PT)])
    plsc.subcore_barrier()

    rows0, rows1 = rowsv.at[0], rowsv.at[1]
    sem0, sem1 = sem.at[0], sem.at[1]

    def _gather(j, buf, s_):
        pltpu.async_copy(m_hbm.at[srcv.at[j]], buf, s_)

    def _drain(buf, s_):
        pltpu.make_async_copy(m_hbm.at[srcv.at[0]], buf, s_).wait()

    def _scatter(j, buf):
        pltpu.sync_copy(buf, aggsh.at[dstv.at[j]], add=True)

    @pl.loop(0, CHUNKS_PW // HALF)
    def _half(half):
        base = s * CHUNKS_PW + half * HALF
        pltpu.sync_copy(src_hbm.at[pl.ds(base, HALF)], srcv)
        pltpu.sync_copy(dst_hbm.at[pl.ds(base, HALF)], dstv)

        _gather(0, rows0, sem0)

        @pl.loop(0, HALF // 2 - 1)
        def _pair(k):
            j = 2 * k
            _gather(j + 1, rows1, sem1)
            _drain(rows0, sem0)
            _scatter(j, rows0)
            _gather(j + 2, rows0, sem0)
            _drain(rows1, sem1)
            _scatter(j + 1, rows1)

        _gather(HALF - 1, rows1, sem1)
        _drain(rows0, sem0)
        _scatter(HALF - 2, rows0)
        _drain(rows1, sem1)
        _scatter(HALF - 1, rows1)

    plsc.subcore_barrier()
    pltpu.sync_copy(aggsh.at[pl.ds(s * ROWS_PT, ROWS_PT)],
                    out_hbm.at[0, pl.ds(s * ROWS_PT, ROWS_PT)])


@functools.cache
def _get_sc_segsum():
    return pl.kernel(
        _sc_segsum_body,
        out_type=jax.ShapeDtypeStruct((AGG_PARTS, NPAD, F), jnp.float32),
        mesh=plsc.VectorSubcoreMesh(core_axis_name="c", subcore_axis_name="s",
                                    num_cores=NCORES),
        scratch_types=[
            pltpu.VMEM((HALF, CHUNK), jnp.int32),
            pltpu.VMEM((HALF, CHUNK), jnp.int32),
            pltpu.VMEM((2, CHUNK, F), jnp.float32),
            pltpu.VMEM_SHARED((NPAD, F), jnp.float32),
            pltpu.SemaphoreType.DMA((2,)),
        ],
    )


def _embed_call(x, w0t, conv0):
    return pl.pallas_call(
        _embed_body,
        grid=(GRID,),
        in_specs=[
            pl.BlockSpec((NB, F), lambda i: (i, 0)),
            pl.BlockSpec((F, F), lambda i: (0, 0)),
            pl.BlockSpec((F, F), lambda i: (0, 0)),
        ],
        out_specs=[
            pl.BlockSpec((NB, F), lambda i: (i, 0)),
            pl.BlockSpec((NB, F), lambda i: (i, 0)),
        ],
        out_shape=[
            jax.ShapeDtypeStruct((N, F), jnp.float32),
            jax.ShapeDtypeStruct((N, F), jnp.float32),
        ],
    )(x, w0t, conv0)


def _gru_call(aggs, h, wih, whh, bih, bhh, conv, with_conv):
    full = lambda i: (0, 0)
    blk = lambda i: (i, 0)
    out_shape = [jax.ShapeDtypeStruct((N, F), jnp.float32)]
    out_specs = [pl.BlockSpec((NB, F), blk)]
    if with_conv:
        out_shape.append(jax.ShapeDtypeStruct((N, F), jnp.float32))
        out_specs.append(pl.BlockSpec((NB, F), blk))
    return pl.pallas_call(
        functools.partial(_gru_body, with_conv=with_conv),
        grid=(GRID,),
        in_specs=[pl.BlockSpec((NB, F), blk)] * AGG_PARTS + [
            pl.BlockSpec((NB, F), blk),
            pl.BlockSpec((F, 3 * F), full),
            pl.BlockSpec((F, 3 * F), full),
            pl.BlockSpec((1, 3 * F), full),
            pl.BlockSpec((1, 3 * F), full),
            pl.BlockSpec((F, F), full),
        ],
        out_specs=out_specs,
        out_shape=out_shape,
    )(*aggs, h, wih, whh, bih, bhh, conv)


def _head_sample_call(h3, w1, b1, w2, b2, noise):
    return pl.pallas_call(
        _head_sample_body,
        out_shape=jax.ShapeDtypeStruct((G, NPER), jnp.float32),
    )(h3, w1, b1, w2, b2, noise)


def kernel(x, edge_index, batch, num_graphs, W0, conv_weight, gru_w_ih,
           gru_w_hh, gru_b_ih, gru_b_hh, w1, b1, w2, b2):
    w0t = W0.T
    wih = gru_w_ih.T
    whh = gru_w_hh.T
    bih = gru_b_ih.reshape(1, 3 * F)
    bhh = gru_b_hh.reshape(1, 3 * F)

    src = edge_index[0]
    dst = edge_index[1]
    pad = NCHUNKS_ALLOC * CHUNK - E
    pad_i = jnp.arange(pad, dtype=jnp.int32)
    src_p = jnp.concatenate([src, pad_i % N])
    dst_p = jnp.concatenate([dst, N + pad_i % (NPAD - N)])
    src2 = src_p.reshape(NCHUNKS_ALLOC, CHUNK)
    dst2 = dst_p.reshape(NCHUNKS_ALLOC, CHUNK)
    zeros = jnp.zeros((NPAD, F), jnp.float32)

    h, m = _embed_call(x, w0t, conv_weight[0])
    sc_segsum = _get_sc_segsum()
    for i in range(3):
        parts = sc_segsum(m, src2, dst2, zeros)
        aggs = [parts[k, :N] for k in range(AGG_PARTS)]
        with_conv = i < 2
        conv_next = conv_weight[i + 1] if with_conv else conv_weight[0]
        res = _gru_call(aggs, h, wih, whh, bih, bhh, conv_next, with_conv)
        if with_conv:
            h, m = res
        else:
            h = res[0] if isinstance(res, (list, tuple)) else res

    h3 = h.reshape(G, NPER, F)
    noise = jax.random.normal(jax.random.key(42), (G, NPER - 1), jnp.float32)
    noise_p = jnp.pad(noise, ((0, 0), (0, 1)))
    pred = _head_sample_call(h3, w1.reshape(1, F), b1.reshape(1, 1),
                             w2.reshape(1, F), b2.reshape(1, 1), noise_p)
    return pred.reshape(-1)


# speedup vs baseline: 3.0728x; 1.3974x over previous
import functools

import jax
import jax.numpy as jnp
from jax import lax
from jax.experimental import pallas as pl
from jax.experimental.pallas import tpu as pltpu
from jax.experimental.pallas import tpu_sc as plsc

N = 10000
E = 160000
F = 128
G = 100
NPER = 100
NB = 1000
GRID = N // NB

NCORES = 2
AGG_PARTS = 2
CHUNK = 128
CHUNKS_PW = 40
NCHUNKS_ALLOC = 32 * CHUNKS_PW
EPAD = NCHUNKS_ALLOC * CHUNK
NPAD = 10112
ROWS_PT = NPAD // 16


def _embed_body(x_ref, w0t_ref, conv0_ref, h_ref, m_ref):
    h = jax.nn.sigmoid(jnp.dot(x_ref[...], w0t_ref[...],
                               preferred_element_type=jnp.float32))
    h_ref[...] = h
    m_ref[...] = jnp.dot(h, conv0_ref[...], preferred_element_type=jnp.float32)


def _gru_body(*refs, with_conv):
    aggs = refs[:AGG_PARTS]
    (h_ref, wih_ref, whh_ref, bih_ref, bhh_ref, conv_ref, h_out_ref,
     *rest) = refs[AGG_PARTS:]
    m_ref = rest[0] if with_conv else None
    agg = aggs[0][...]
    for a in aggs[1:]:
        agg = agg + a[...]
    h = h_ref[...]
    gi = jnp.dot(agg, wih_ref[...], preferred_element_type=jnp.float32) + bih_ref[...]
    gh = jnp.dot(h, whh_ref[...], preferred_element_type=jnp.float32) + bhh_ref[...]
    r = jax.nn.sigmoid(gi[:, :F] + gh[:, :F])
    z = jax.nn.sigmoid(gi[:, F:2 * F] + gh[:, F:2 * F])
    n = jnp.tanh(gi[:, 2 * F:] + r * gh[:, 2 * F:])
    hn = (1.0 - z) * n + z * h
    h_out_ref[...] = hn
    if with_conv:
        m_ref[...] = jnp.dot(hn, conv_ref[...], preferred_element_type=jnp.float32)


def _softplus(x):
    return jnp.maximum(x, 0.0) + jnp.log1p(jnp.exp(-jnp.abs(x)))


def _head_sample_body(h3_ref, w1_ref, b1_ref, w2_ref, b2_ref, noise_ref, out_ref):
    h3 = jnp.maximum(h3_ref[...], 0.0)
    w1 = w1_ref[...].reshape(1, 1, F)
    w2 = w2_ref[...].reshape(1, 1, F)
    mu = jnp.sum(h3 * w1, axis=2) + b1_ref[0, 0]
    sigma = _softplus(jnp.sum(h3 * w2, axis=2) + b2_ref[0, 0])

    col = lax.broadcasted_iota(jnp.int32, (G, NPER), 1)
    row_t = lax.broadcasted_iota(jnp.int32, (NPER, NPER), 0)
    col_t = lax.broadcasted_iota(jnp.int32, (NPER, NPER), 1)
    main = col < (NPER - 1)
    lastc = col == (NPER - 1)

    s = jnp.where(main, sigma, 0.0)
    mus = jnp.where(main, mu, 0.0)
    sn = jnp.sum(jnp.where(lastc, sigma, 0.0), axis=1, keepdims=True)
    mun = jnp.sum(jnp.where(lastc, mu, 0.0), axis=1, keepdims=True)

    d = s + 1e-6
    sum_s = jnp.sum(s, axis=1, keepdims=True)
    sum_mu = jnp.sum(mus, axis=1, keepdims=True)
    tot = sn + sum_s
    c0 = 1.0 / tot
    c = -mun / sn
    rmean = c * s + mus - c0 * (c * sum_s + sum_mu) * s

    t_rev = (row_t >= col_t).astype(jnp.float32)
    t_ex = (row_t < col_t).astype(jnp.float32)
    rev = jnp.dot(s, t_rev, preferred_element_type=jnp.float32)
    cex = jnp.dot(s / d, t_ex, preferred_element_type=jnp.float32)
    t = 1.0 / (-(sn + rev + 1e-6 * cex))
    ell = jnp.sqrt(d + t * s * s)
    w = t * s / ell

    noise = noise_ref[...]
    wn = w * noise
    cum_wn = jnp.dot(wn, t_ex, preferred_element_type=jnp.float32)
    xr = rmean + ell * noise + s * cum_wn
    xr = jnp.where(main, xr, 0.0)
    last = -jnp.sum(xr, axis=1, keepdims=True)
    out_ref[...] = jnp.where(lastc, jnp.broadcast_to(last, (G, NPER)), xr)


def _sc_segsum_body(m_hbm, src_hbm, dst_hbm, zeros_hbm, out_hbm,
                    srcv, dstv, rowsv, aggsh, sem):
    c = lax.axis_index("c")
    s = lax.axis_index("s")
    wid = s * NCORES + c

    pltpu.sync_copy(zeros_hbm.at[pl.ds(s * ROWS_PT, ROWS_PT)],
                    aggsh.at[pl.ds(s * ROWS_PT, ROWS_PT)])

    pltpu.sync_copy(src_hbm.at[pl.ds(wid * CHUNKS_PW, CHUNKS_PW)], srcv)
    pltpu.sync_copy(dst_hbm.at[pl.ds(wid * CHUNKS_PW, CHUNKS_PW)], dstv)
    plsc.subcore_barrier()

    rows0, rows1 = rowsv.at[0], rowsv.at[1]
    sem0, sem1 = sem.at[0], sem.at[1]

    def _gather(j, buf, s_):
        pltpu.async_copy(m_hbm.at[srcv.at[j]], buf, s_)

    def _drain(buf, s_):
        pltpu.make_async_copy(m_hbm.at[srcv.at[0]], buf, s_).wait()

    def _scatter(j, buf):
        pltpu.sync_copy(buf, aggsh.at[dstv.at[j]], add=True)

    _gather(0, rows0, sem0)

    @pl.loop(0, CHUNKS_PW // 2 - 1)
    def _pair(k):
        j = 2 * k
        _gather(j + 1, rows1, sem1)
        _drain(rows0, sem0)
        _scatter(j, rows0)
        _gather(j + 2, rows0, sem0)
        _drain(rows1, sem1)
        _scatter(j + 1, rows1)

    _gather(CHUNKS_PW - 1, rows1, sem1)
    _drain(rows0, sem0)
    _scatter(CHUNKS_PW - 2, rows0)
    _drain(rows1, sem1)
    _scatter(CHUNKS_PW - 1, rows1)

    plsc.subcore_barrier()
    pltpu.sync_copy(aggsh.at[pl.ds(s * ROWS_PT, ROWS_PT)],
                    out_hbm.at[c, pl.ds(s * ROWS_PT, ROWS_PT)])


@functools.cache
def _get_sc_segsum():
    return pl.kernel(
        _sc_segsum_body,
        out_type=jax.ShapeDtypeStruct((AGG_PARTS, NPAD, F), jnp.float32),
        mesh=plsc.VectorSubcoreMesh(core_axis_name="c", subcore_axis_name="s",
                                    num_cores=NCORES),
        scratch_types=[
            pltpu.VMEM((CHUNKS_PW, CHUNK), jnp.int32),
            pltpu.VMEM((CHUNKS_PW, CHUNK), jnp.int32),
            pltpu.VMEM((2, CHUNK, F), jnp.float32),
            pltpu.VMEM_SHARED((NPAD, F), jnp.float32),
            pltpu.SemaphoreType.DMA((2,)),
        ],
    )


def _embed_call(x, w0t, conv0):
    return pl.pallas_call(
        _embed_body,
        grid=(GRID,),
        in_specs=[
            pl.BlockSpec((NB, F), lambda i: (i, 0)),
            pl.BlockSpec((F, F), lambda i: (0, 0)),
            pl.BlockSpec((F, F), lambda i: (0, 0)),
        ],
        out_specs=[
            pl.BlockSpec((NB, F), lambda i: (i, 0)),
            pl.BlockSpec((NB, F), lambda i: (i, 0)),
        ],
        out_shape=[
            jax.ShapeDtypeStruct((N, F), jnp.float32),
            jax.ShapeDtypeStruct((N, F), jnp.float32),
        ],
    )(x, w0t, conv0)


def _gru_call(aggs, h, wih, whh, bih, bhh, conv, with_conv):
    full = lambda i: (0, 0)
    blk = lambda i: (i, 0)
    out_shape = [jax.ShapeDtypeStruct((N, F), jnp.float32)]
    out_specs = [pl.BlockSpec((NB, F), blk)]
    if with_conv:
        out_shape.append(jax.ShapeDtypeStruct((N, F), jnp.float32))
        out_specs.append(pl.BlockSpec((NB, F), blk))
    return pl.pallas_call(
        functools.partial(_gru_body, with_conv=with_conv),
        grid=(GRID,),
        in_specs=[pl.BlockSpec((NB, F), blk)] * AGG_PARTS + [
            pl.BlockSpec((NB, F), blk),
            pl.BlockSpec((F, 3 * F), full),
            pl.BlockSpec((F, 3 * F), full),
            pl.BlockSpec((1, 3 * F), full),
            pl.BlockSpec((1, 3 * F), full),
            pl.BlockSpec((F, F), full),
        ],
        out_specs=out_specs,
        out_shape=out_shape,
    )(*aggs, h, wih, whh, bih, bhh, conv)


def _head_sample_call(h3, w1, b1, w2, b2, noise):
    return pl.pallas_call(
        _head_sample_body,
        out_shape=jax.ShapeDtypeStruct((G, NPER), jnp.float32),
    )(h3, w1, b1, w2, b2, noise)


def kernel(x, edge_index, batch, num_graphs, W0, conv_weight, gru_w_ih,
           gru_w_hh, gru_b_ih, gru_b_hh, w1, b1, w2, b2):
    w0t = W0.T
    wih = gru_w_ih.T
    whh = gru_w_hh.T
    bih = gru_b_ih.reshape(1, 3 * F)
    bhh = gru_b_hh.reshape(1, 3 * F)

    src = edge_index[0]
    dst = edge_index[1]
    pad = NCHUNKS_ALLOC * CHUNK - E
    pad_i = jnp.arange(pad, dtype=jnp.int32)
    src_p = jnp.concatenate([src, pad_i % N])
    dst_p = jnp.concatenate([dst, N + pad_i % (NPAD - N)])
    src2 = src_p.reshape(NCHUNKS_ALLOC, CHUNK)
    dst2 = dst_p.reshape(NCHUNKS_ALLOC, CHUNK)
    zeros = jnp.zeros((NPAD, F), jnp.float32)

    h, m = _embed_call(x, w0t, conv_weight[0])
    sc_segsum = _get_sc_segsum()
    for i in range(3):
        parts = sc_segsum(m, src2, dst2, zeros)
        aggs = [parts[k, :N] for k in range(AGG_PARTS)]
        with_conv = i < 2
        conv_next = conv_weight[i + 1] if with_conv else conv_weight[0]
        res = _gru_call(aggs, h, wih, whh, bih, bhh, conv_next, with_conv)
        if with_conv:
            h, m = res
        else:
            h = res[0] if isinstance(res, (list, tuple)) else res

    h3 = h.reshape(G, NPER, F)
    noise = jax.random.normal(jax.random.key(42), (G, NPER - 1), jnp.float32)
    noise_p = jnp.pad(noise, ((0, 0), (0, 1)))
    pred = _head_sample_call(h3, w1.reshape(1, F), b1.reshape(1, 1),
                             w2.reshape(1, F), b2.reshape(1, 1), noise_p)
    return pred.reshape(-1)


# speedup vs baseline: 3.3037x; 1.0751x over previous
import functools

import jax
import jax.numpy as jnp
from jax import lax
from jax.experimental import pallas as pl
from jax.experimental.pallas import tpu as pltpu
from jax.experimental.pallas import tpu_sc as plsc

N = 10000
E = 160000
F = 128
G = 100
NPER = 100
NB = 1000
GRID = N // NB

NCORES = 2
AGG_PARTS = 2
CHUNK = 128
CHUNKS_PW = 40
NCHUNKS_ALLOC = 32 * CHUNKS_PW
EPAD = NCHUNKS_ALLOC * CHUNK
NPAD = 10112
ROWS_PT = NPAD // 16


def _dot_t(a, b):
    return lax.dot_general(a, b, (((1,), (1,)), ((), ())),
                           preferred_element_type=jnp.float32)


def _embed_body(x_ref, w0_ref, conv0_ref, h_ref, m_ref):
    h = jax.nn.sigmoid(_dot_t(x_ref[...], w0_ref[...]))
    h_ref[...] = h
    m_ref[...] = jnp.dot(h, conv0_ref[...], preferred_element_type=jnp.float32)


def _gru_body(parts_ref, h_ref, wih_ref, whh_ref, bih_ref, bhh_ref,
              conv_ref, h_out_ref, m_ref=None, *, with_conv):
    agg = parts_ref[0] + parts_ref[1]
    h = h_ref[...]
    gi = _dot_t(agg, wih_ref[...]) + bih_ref[...]
    gh = _dot_t(h, whh_ref[...]) + bhh_ref[...]
    r = jax.nn.sigmoid(gi[:, :F] + gh[:, :F])
    z = jax.nn.sigmoid(gi[:, F:2 * F] + gh[:, F:2 * F])
    n = jnp.tanh(gi[:, 2 * F:] + r * gh[:, 2 * F:])
    hn = (1.0 - z) * n + z * h
    h_out_ref[...] = hn
    if with_conv:
        m_ref[...] = jnp.dot(hn, conv_ref[...], preferred_element_type=jnp.float32)


def _softplus(x):
    return jnp.maximum(x, 0.0) + jnp.log1p(jnp.exp(-jnp.abs(x)))


def _head_sample_body(h3_ref, w1_ref, b1_ref, w2_ref, b2_ref, noise_ref, out_ref):
    h3 = jnp.maximum(h3_ref[...], 0.0)
    w1 = w1_ref[...].reshape(1, 1, F)
    w2 = w2_ref[...].reshape(1, 1, F)
    mu = jnp.sum(h3 * w1, axis=2) + b1_ref[0, 0]
    sigma = _softplus(jnp.sum(h3 * w2, axis=2) + b2_ref[0, 0])

    col = lax.broadcasted_iota(jnp.int32, (G, NPER), 1)
    row_t = lax.broadcasted_iota(jnp.int32, (NPER, NPER), 0)
    col_t = lax.broadcasted_iota(jnp.int32, (NPER, NPER), 1)
    main = col < (NPER - 1)
    lastc = col == (NPER - 1)

    s = jnp.where(main, sigma, 0.0)
    mus = jnp.where(main, mu, 0.0)
    sn = jnp.sum(jnp.where(lastc, sigma, 0.0), axis=1, keepdims=True)
    mun = jnp.sum(jnp.where(lastc, mu, 0.0), axis=1, keepdims=True)

    d = s + 1e-6
    sum_s = jnp.sum(s, axis=1, keepdims=True)
    sum_mu = jnp.sum(mus, axis=1, keepdims=True)
    tot = sn + sum_s
    c0 = 1.0 / tot
    c = -mun / sn
    rmean = c * s + mus - c0 * (c * sum_s + sum_mu) * s

    t_rev = (row_t >= col_t).astype(jnp.float32)
    t_ex = (row_t < col_t).astype(jnp.float32)
    rev = jnp.dot(s, t_rev, preferred_element_type=jnp.float32)
    cex = jnp.dot(s / d, t_ex, preferred_element_type=jnp.float32)
    t = 1.0 / (-(sn + rev + 1e-6 * cex))
    ell = jnp.sqrt(d + t * s * s)
    w = t * s / ell

    noise = noise_ref[...]
    wn = w * noise
    cum_wn = jnp.dot(wn, t_ex, preferred_element_type=jnp.float32)
    xr = rmean + ell * noise + s * cum_wn
    xr = jnp.where(main, xr, 0.0)
    last = -jnp.sum(xr, axis=1, keepdims=True)
    out_ref[...] = jnp.where(lastc, jnp.broadcast_to(last, (G, NPER)), xr)


def _sc_segsum_body(m_hbm, src_hbm, dst_hbm, zeros_hbm, out_hbm,
                    srcv, dstv, rowsv, aggsh, sem):
    c = lax.axis_index("c")
    s = lax.axis_index("s")
    wid = s * NCORES + c

    pltpu.sync_copy(zeros_hbm.at[pl.ds(s * ROWS_PT, ROWS_PT)],
                    aggsh.at[pl.ds(s * ROWS_PT, ROWS_PT)])

    pltpu.sync_copy(src_hbm.at[pl.ds(wid * CHUNKS_PW, CHUNKS_PW)], srcv)
    pltpu.sync_copy(dst_hbm.at[pl.ds(wid * CHUNKS_PW, CHUNKS_PW)], dstv)
    plsc.subcore_barrier()

    rows0, rows1 = rowsv.at[0], rowsv.at[1]
    sem0, sem1 = sem.at[0], sem.at[1]

    def _gather(j, buf, s_):
        pltpu.async_copy(m_hbm.at[srcv.at[j]], buf, s_)

    def _drain(buf, s_):
        pltpu.make_async_copy(m_hbm.at[srcv.at[0]], buf, s_).wait()

    def _scatter(j, buf):
        pltpu.sync_copy(buf, aggsh.at[dstv.at[j]], add=True)

    _gather(0, rows0, sem0)

    @pl.loop(0, CHUNKS_PW // 2 - 1)
    def _pair(k):
        j = 2 * k
        _gather(j + 1, rows1, sem1)
        _drain(rows0, sem0)
        _scatter(j, rows0)
        _gather(j + 2, rows0, sem0)
        _drain(rows1, sem1)
        _scatter(j + 1, rows1)

    _gather(CHUNKS_PW - 1, rows1, sem1)
    _drain(rows0, sem0)
    _scatter(CHUNKS_PW - 2, rows0)
    _drain(rows1, sem1)
    _scatter(CHUNKS_PW - 1, rows1)

    plsc.subcore_barrier()
    pltpu.sync_copy(aggsh.at[pl.ds(s * ROWS_PT, ROWS_PT)],
                    out_hbm.at[c, pl.ds(s * ROWS_PT, ROWS_PT)])


@functools.cache
def _get_sc_segsum():
    return pl.kernel(
        _sc_segsum_body,
        out_type=jax.ShapeDtypeStruct((AGG_PARTS, NPAD, F), jnp.float32),
        mesh=plsc.VectorSubcoreMesh(core_axis_name="c", subcore_axis_name="s",
                                    num_cores=NCORES),
        scratch_types=[
            pltpu.VMEM((CHUNKS_PW, CHUNK), jnp.int32),
            pltpu.VMEM((CHUNKS_PW, CHUNK), jnp.int32),
            pltpu.VMEM((2, CHUNK, F), jnp.float32),
            pltpu.VMEM_SHARED((NPAD, F), jnp.float32),
            pltpu.SemaphoreType.DMA((2,)),
        ],
    )


def _embed_call(x, w0t, conv0):
    return pl.pallas_call(
        _embed_body,
        grid=(GRID,),
        in_specs=[
            pl.BlockSpec((NB, F), lambda i: (i, 0)),
            pl.BlockSpec((F, F), lambda i: (0, 0)),
            pl.BlockSpec((F, F), lambda i: (0, 0)),
        ],
        out_specs=[
            pl.BlockSpec((NB, F), lambda i: (i, 0)),
            pl.BlockSpec((NB, F), lambda i: (i, 0)),
        ],
        out_shape=[
            jax.ShapeDtypeStruct((N, F), jnp.float32),
            jax.ShapeDtypeStruct((N, F), jnp.float32),
        ],
    )(x, w0t, conv0)


def _gru_call(parts, h, wih, whh, bih, bhh, conv, with_conv):
    full = lambda i: (0, 0)
    blk = lambda i: (i, 0)
    out_shape = [jax.ShapeDtypeStruct((N, F), jnp.float32)]
    out_specs = [pl.BlockSpec((NB, F), blk)]
    if with_conv:
        out_shape.append(jax.ShapeDtypeStruct((N, F), jnp.float32))
        out_specs.append(pl.BlockSpec((NB, F), blk))
    return pl.pallas_call(
        functools.partial(_gru_body, with_conv=with_conv),
        grid=(GRID,),
        in_specs=[
            pl.BlockSpec((AGG_PARTS, NB, F), lambda i: (0, i, 0)),
            pl.BlockSpec((NB, F), blk),
            pl.BlockSpec((3 * F, F), full),
            pl.BlockSpec((3 * F, F), full),
            pl.BlockSpec((1, 3 * F), full),
            pl.BlockSpec((1, 3 * F), full),
            pl.BlockSpec((F, F), full),
        ],
        out_specs=out_specs,
        out_shape=out_shape,
    )(parts, h, wih, whh, bih, bhh, conv)


def _head_sample_call(h3, w1, b1, w2, b2, noise):
    return pl.pallas_call(
        _head_sample_body,
        out_shape=jax.ShapeDtypeStruct((G, NPER), jnp.float32),
    )(h3, w1, b1, w2, b2, noise)


def kernel(x, edge_index, batch, num_graphs, W0, conv_weight, gru_w_ih,
           gru_w_hh, gru_b_ih, gru_b_hh, w1, b1, w2, b2):
    bih = gru_b_ih.reshape(1, 3 * F)
    bhh = gru_b_hh.reshape(1, 3 * F)

    src = edge_index[0]
    dst = edge_index[1]
    pad = NCHUNKS_ALLOC * CHUNK - E
    pad_i = jnp.arange(pad, dtype=jnp.int32)
    src_p = jnp.concatenate([src, pad_i % N])
    dst_p = jnp.concatenate([dst, N + pad_i % (NPAD - N)])
    src2 = src_p.reshape(NCHUNKS_ALLOC, CHUNK)
    dst2 = dst_p.reshape(NCHUNKS_ALLOC, CHUNK)
    zeros = jnp.zeros((NPAD, F), jnp.float32)

    h, m = _embed_call(x, W0, conv_weight[0])
    sc_segsum = _get_sc_segsum()
    for i in range(3):
        parts = sc_segsum(m, src2, dst2, zeros)
        with_conv = i < 2
        conv_next = conv_weight[i + 1] if with_conv else conv_weight[0]
        res = _gru_call(parts, h, gru_w_ih, gru_w_hh, bih, bhh,
                        conv_next, with_conv)
        if with_conv:
            h, m = res
        else:
            h = res[0] if isinstance(res, (list, tuple)) else res

    h3 = h.reshape(G, NPER, F)
    noise = jax.random.normal(jax.random.key(42), (G, NPER - 1), jnp.float32)
    noise_p = jnp.pad(noise, ((0, 0), (0, 1)))
    pred = _head_sample_call(h3, w1.reshape(1, F), b1.reshape(1, 1),
                             w2.reshape(1, F), b2.reshape(1, 1), noise_p)
    return pred.reshape(-1)


# speedup vs baseline: 3.4256x; 1.0369x over previous
import functools

import jax
import jax.numpy as jnp
from jax import lax
from jax.experimental import pallas as pl
from jax.experimental.pallas import tpu as pltpu
from jax.experimental.pallas import tpu_sc as plsc

N = 10000
E = 160000
F = 128
G = 100
NPER = 100
NB = 2000
GRID = N // NB

NCORES = 2
AGG_PARTS = 2
CHUNK = 128
CHUNKS_PW = 40
NCHUNKS_ALLOC = 32 * CHUNKS_PW
EPAD = NCHUNKS_ALLOC * CHUNK
NPAD = 10112
ROWS_PT = NPAD // 16


def _dot_t(a, b):
    return lax.dot_general(a, b, (((1,), (1,)), ((), ())),
                           preferred_element_type=jnp.float32)


def _embed_body(x_ref, w0_ref, conv0_ref, h_ref, m_ref):
    h = jax.nn.sigmoid(_dot_t(x_ref[...], w0_ref[...]))
    h_ref[...] = h
    m_ref[...] = jnp.dot(h, conv0_ref[...], preferred_element_type=jnp.float32)


def _gru_body(parts_ref, h_ref, wih_ref, whh_ref, bih_ref, bhh_ref,
              conv_ref, h_out_ref, m_ref=None, *, with_conv):
    agg = parts_ref[0] + parts_ref[1]
    h = h_ref[...]
    gi = _dot_t(agg, wih_ref[...]) + bih_ref[...]
    gh = _dot_t(h, whh_ref[...]) + bhh_ref[...]
    r = jax.nn.sigmoid(gi[:, :F] + gh[:, :F])
    z = jax.nn.sigmoid(gi[:, F:2 * F] + gh[:, F:2 * F])
    n = jnp.tanh(gi[:, 2 * F:] + r * gh[:, 2 * F:])
    hn = (1.0 - z) * n + z * h
    h_out_ref[...] = hn
    if with_conv:
        m_ref[...] = jnp.dot(hn, conv_ref[...], preferred_element_type=jnp.float32)


def _softplus(x):
    return jnp.maximum(x, 0.0) + jnp.log1p(jnp.exp(-jnp.abs(x)))


def _head_sample_body(h3_ref, w1_ref, b1_ref, w2_ref, b2_ref, noise_ref, out_ref):
    h3 = jnp.maximum(h3_ref[...], 0.0)
    w1 = w1_ref[...].reshape(1, 1, F)
    w2 = w2_ref[...].reshape(1, 1, F)
    mu = jnp.sum(h3 * w1, axis=2) + b1_ref[0, 0]
    sigma = _softplus(jnp.sum(h3 * w2, axis=2) + b2_ref[0, 0])

    col = lax.broadcasted_iota(jnp.int32, (G, NPER), 1)
    row_t = lax.broadcasted_iota(jnp.int32, (NPER, NPER), 0)
    col_t = lax.broadcasted_iota(jnp.int32, (NPER, NPER), 1)
    main = col < (NPER - 1)
    lastc = col == (NPER - 1)

    s = jnp.where(main, sigma, 0.0)
    mus = jnp.where(main, mu, 0.0)
    sn = jnp.sum(jnp.where(lastc, sigma, 0.0), axis=1, keepdims=True)
    mun = jnp.sum(jnp.where(lastc, mu, 0.0), axis=1, keepdims=True)

    d = s + 1e-6
    sum_s = jnp.sum(s, axis=1, keepdims=True)
    sum_mu = jnp.sum(mus, axis=1, keepdims=True)
    tot = sn + sum_s
    c0 = 1.0 / tot
    c = -mun / sn
    rmean = c * s + mus - c0 * (c * sum_s + sum_mu) * s

    t_rev = (row_t >= col_t).astype(jnp.float32)
    t_ex = (row_t < col_t).astype(jnp.float32)
    rev = jnp.dot(s, t_rev, preferred_element_type=jnp.float32)
    cex = jnp.dot(s / d, t_ex, preferred_element_type=jnp.float32)
    t = 1.0 / (-(sn + rev + 1e-6 * cex))
    ell = jnp.sqrt(d + t * s * s)
    w = t * s / ell

    noise = noise_ref[...]
    wn = w * noise
    cum_wn = jnp.dot(wn, t_ex, preferred_element_type=jnp.float32)
    xr = rmean + ell * noise + s * cum_wn
    xr = jnp.where(main, xr, 0.0)
    last = -jnp.sum(xr, axis=1, keepdims=True)
    out_ref[...] = jnp.where(lastc, jnp.broadcast_to(last, (G, NPER)), xr)


def _sc_segsum_body(m_hbm, src_hbm, dst_hbm, zeros_hbm, out_hbm,
                    srcv, dstv, rowsv, aggsh, sem):
    c = lax.axis_index("c")
    s = lax.axis_index("s")
    wid = s * NCORES + c

    pltpu.sync_copy(zeros_hbm.at[pl.ds(s * ROWS_PT, ROWS_PT)],
                    aggsh.at[pl.ds(s * ROWS_PT, ROWS_PT)])

    pltpu.sync_copy(src_hbm.at[pl.ds(wid * CHUNKS_PW, CHUNKS_PW)], srcv)
    pltpu.sync_copy(dst_hbm.at[pl.ds(wid * CHUNKS_PW, CHUNKS_PW)], dstv)
    plsc.subcore_barrier()

    rows0, rows1 = rowsv.at[0], rowsv.at[1]
    sem0, sem1 = sem.at[0], sem.at[1]

    def _gather(j, buf, s_):
        pltpu.async_copy(m_hbm.at[srcv.at[j]], buf, s_)

    def _drain(buf, s_):
        pltpu.make_async_copy(m_hbm.at[srcv.at[0]], buf, s_).wait()

    def _scatter(j, buf):
        pltpu.sync_copy(buf, aggsh.at[dstv.at[j]], add=True)

    _gather(0, rows0, sem0)

    @pl.loop(0, CHUNKS_PW // 2 - 1)
    def _pair(k):
        j = 2 * k
        _gather(j + 1, rows1, sem1)
        _drain(rows0, sem0)
        _scatter(j, rows0)
        _gather(j + 2, rows0, sem0)
        _drain(rows1, sem1)
        _scatter(j + 1, rows1)

    _gather(CHUNKS_PW - 1, rows1, sem1)
    _drain(rows0, sem0)
    _scatter(CHUNKS_PW - 2, rows0)
    _drain(rows1, sem1)
    _scatter(CHUNKS_PW - 1, rows1)

    plsc.subcore_barrier()
    pltpu.sync_copy(aggsh.at[pl.ds(s * ROWS_PT, ROWS_PT)],
                    out_hbm.at[c, pl.ds(s * ROWS_PT, ROWS_PT)])


@functools.cache
def _get_sc_segsum():
    return pl.kernel(
        _sc_segsum_body,
        out_type=jax.ShapeDtypeStruct((AGG_PARTS, NPAD, F), jnp.float32),
        mesh=plsc.VectorSubcoreMesh(core_axis_name="c", subcore_axis_name="s",
                                    num_cores=NCORES),
        scratch_types=[
            pltpu.VMEM((CHUNKS_PW, CHUNK), jnp.int32),
            pltpu.VMEM((CHUNKS_PW, CHUNK), jnp.int32),
            pltpu.VMEM((2, CHUNK, F), jnp.float32),
            pltpu.VMEM_SHARED((NPAD, F), jnp.float32),
            pltpu.SemaphoreType.DMA((2,)),
        ],
    )


def _embed_call(x, w0t, conv0):
    return pl.pallas_call(
        _embed_body,
        grid=(GRID,),
        in_specs=[
            pl.BlockSpec((NB, F), lambda i: (i, 0)),
            pl.BlockSpec((F, F), lambda i: (0, 0)),
            pl.BlockSpec((F, F), lambda i: (0, 0)),
        ],
        out_specs=[
            pl.BlockSpec((NB, F), lambda i: (i, 0)),
            pl.BlockSpec((NB, F), lambda i: (i, 0)),
        ],
        out_shape=[
            jax.ShapeDtypeStruct((N, F), jnp.float32),
            jax.ShapeDtypeStruct((N, F), jnp.float32),
        ],
    )(x, w0t, conv0)


def _gru_call(parts, h, wih, whh, bih, bhh, conv, with_conv):
    full = lambda i: (0, 0)
    blk = lambda i: (i, 0)
    out_shape = [jax.ShapeDtypeStruct((N, F), jnp.float32)]
    out_specs = [pl.BlockSpec((NB, F), blk)]
    if with_conv:
        out_shape.append(jax.ShapeDtypeStruct((N, F), jnp.float32))
        out_specs.append(pl.BlockSpec((NB, F), blk))
    return pl.pallas_call(
        functools.partial(_gru_body, with_conv=with_conv),
        grid=(GRID,),
        in_specs=[
            pl.BlockSpec((AGG_PARTS, NB, F), lambda i: (0, i, 0)),
            pl.BlockSpec((NB, F), blk),
            pl.BlockSpec((3 * F, F), full),
            pl.BlockSpec((3 * F, F), full),
            pl.BlockSpec((1, 3 * F), full),
            pl.BlockSpec((1, 3 * F), full),
            pl.BlockSpec((F, F), full),
        ],
        out_specs=out_specs,
        out_shape=out_shape,
    )(parts, h, wih, whh, bih, bhh, conv)


def _head_sample_call(h3, w1, b1, w2, b2, noise):
    return pl.pallas_call(
        _head_sample_body,
        out_shape=jax.ShapeDtypeStruct((G, NPER), jnp.float32),
    )(h3, w1, b1, w2, b2, noise)


def kernel(x, edge_index, batch, num_graphs, W0, conv_weight, gru_w_ih,
           gru_w_hh, gru_b_ih, gru_b_hh, w1, b1, w2, b2):
    bih = gru_b_ih.reshape(1, 3 * F)
    bhh = gru_b_hh.reshape(1, 3 * F)

    src = edge_index[0]
    dst = edge_index[1]
    pad = NCHUNKS_ALLOC * CHUNK - E
    pad_i = jnp.arange(pad, dtype=jnp.int32)
    src_p = jnp.concatenate([src, pad_i % N])
    dst_p = jnp.concatenate([dst, N + pad_i % (NPAD - N)])
    src2 = src_p.reshape(NCHUNKS_ALLOC, CHUNK)
    dst2 = dst_p.reshape(NCHUNKS_ALLOC, CHUNK)
    zeros = jnp.zeros((NPAD, F), jnp.float32)

    h, m = _embed_call(x, W0, conv_weight[0])
    sc_segsum = _get_sc_segsum()
    for i in range(3):
        parts = sc_segsum(m, src2, dst2, zeros)
        with_conv = i < 2
        conv_next = conv_weight[i + 1] if with_conv else conv_weight[0]
        res = _gru_call(parts, h, gru_w_ih, gru_w_hh, bih, bhh,
                        conv_next, with_conv)
        if with_conv:
            h, m = res
        else:
            h = res[0] if isinstance(res, (list, tuple)) else res

    h3 = h.reshape(G, NPER, F)
    noise = jax.random.normal(jax.random.key(42), (G, NPER - 1), jnp.float32)
    noise_p = jnp.pad(noise, ((0, 0), (0, 1)))
    pred = _head_sample_call(h3, w1.reshape(1, F), b1.reshape(1, 1),
                             w2.reshape(1, F), b2.reshape(1, 1), noise_p)
    return pred.reshape(-1)


# speedup vs baseline: 3.4606x; 1.0102x over previous
import functools

import jax
import jax.numpy as jnp
from jax import lax
from jax.experimental import pallas as pl
from jax.experimental.pallas import tpu as pltpu
from jax.experimental.pallas import tpu_sc as plsc

N = 10000
E = 160000
F = 128
G = 100
NPER = 100
NB = 5000
GRID = N // NB

NCORES = 2
AGG_PARTS = 2
CHUNK = 128
CHUNKS_PW = 40
NCHUNKS_ALLOC = 32 * CHUNKS_PW
EPAD = NCHUNKS_ALLOC * CHUNK
NPAD = 10112
ROWS_PT = NPAD // 16


def _dot_t(a, b):
    return lax.dot_general(a, b, (((1,), (1,)), ((), ())),
                           preferred_element_type=jnp.float32)


def _embed_body(x_ref, w0_ref, conv0_ref, h_ref, m_ref):
    h = jax.nn.sigmoid(_dot_t(x_ref[...], w0_ref[...]))
    h_ref[...] = h
    m_ref[...] = jnp.dot(h, conv0_ref[...], preferred_element_type=jnp.float32)


def _gru_body(parts_ref, h_ref, wih_ref, whh_ref, bih_ref, bhh_ref,
              conv_ref, h_out_ref, m_ref=None, *, with_conv):
    agg = parts_ref[0] + parts_ref[1]
    h = h_ref[...]
    gi = _dot_t(agg, wih_ref[...]) + bih_ref[...]
    gh = _dot_t(h, whh_ref[...]) + bhh_ref[...]
    r = jax.nn.sigmoid(gi[:, :F] + gh[:, :F])
    z = jax.nn.sigmoid(gi[:, F:2 * F] + gh[:, F:2 * F])
    n = jnp.tanh(gi[:, 2 * F:] + r * gh[:, 2 * F:])
    hn = (1.0 - z) * n + z * h
    h_out_ref[...] = hn
    if with_conv:
        m_ref[...] = jnp.dot(hn, conv_ref[...], preferred_element_type=jnp.float32)


def _softplus(x):
    return jnp.maximum(x, 0.0) + jnp.log1p(jnp.exp(-jnp.abs(x)))


def _head_sample_body(h3_ref, w1_ref, b1_ref, w2_ref, b2_ref, noise_ref, out_ref):
    h3 = jnp.maximum(h3_ref[...], 0.0)
    w1 = w1_ref[...].reshape(1, 1, F)
    w2 = w2_ref[...].reshape(1, 1, F)
    mu = jnp.sum(h3 * w1, axis=2) + b1_ref[0, 0]
    sigma = _softplus(jnp.sum(h3 * w2, axis=2) + b2_ref[0, 0])

    col = lax.broadcasted_iota(jnp.int32, (G, NPER), 1)
    row_t = lax.broadcasted_iota(jnp.int32, (NPER, NPER), 0)
    col_t = lax.broadcasted_iota(jnp.int32, (NPER, NPER), 1)
    main = col < (NPER - 1)
    lastc = col == (NPER - 1)

    s = jnp.where(main, sigma, 0.0)
    mus = jnp.where(main, mu, 0.0)
    sn = jnp.sum(jnp.where(lastc, sigma, 0.0), axis=1, keepdims=True)
    mun = jnp.sum(jnp.where(lastc, mu, 0.0), axis=1, keepdims=True)

    d = s + 1e-6
    sum_s = jnp.sum(s, axis=1, keepdims=True)
    sum_mu = jnp.sum(mus, axis=1, keepdims=True)
    tot = sn + sum_s
    c0 = 1.0 / tot
    c = -mun / sn
    rmean = c * s + mus - c0 * (c * sum_s + sum_mu) * s

    t_rev = (row_t >= col_t).astype(jnp.float32)
    t_ex = (row_t < col_t).astype(jnp.float32)
    rev = jnp.dot(s, t_rev, preferred_element_type=jnp.float32)
    cex = jnp.dot(s / d, t_ex, preferred_element_type=jnp.float32)
    t = 1.0 / (-(sn + rev + 1e-6 * cex))
    ell = jnp.sqrt(d + t * s * s)
    w = t * s / ell

    noise = noise_ref[...]
    wn = w * noise
    cum_wn = jnp.dot(wn, t_ex, preferred_element_type=jnp.float32)
    xr = rmean + ell * noise + s * cum_wn
    xr = jnp.where(main, xr, 0.0)
    last = -jnp.sum(xr, axis=1, keepdims=True)
    out_ref[...] = jnp.where(lastc, jnp.broadcast_to(last, (G, NPER)), xr)


def _sc_segsum_body(m_hbm, src_hbm, dst_hbm, zeros_hbm, out_hbm,
                    srcv, dstv, rowsv, aggsh, sem):
    c = lax.axis_index("c")
    s = lax.axis_index("s")
    wid = s * NCORES + c

    pltpu.sync_copy(zeros_hbm.at[pl.ds(s * ROWS_PT, ROWS_PT)],
                    aggsh.at[pl.ds(s * ROWS_PT, ROWS_PT)])

    pltpu.sync_copy(src_hbm.at[pl.ds(wid * CHUNKS_PW, CHUNKS_PW)], srcv)
    pltpu.sync_copy(dst_hbm.at[pl.ds(wid * CHUNKS_PW, CHUNKS_PW)], dstv)
    plsc.subcore_barrier()

    rows0, rows1 = rowsv.at[0], rowsv.at[1]
    sem0, sem1 = sem.at[0], sem.at[1]

    def _gather(j, buf, s_):
        pltpu.async_copy(m_hbm.at[srcv.at[j]], buf, s_)

    def _drain(buf, s_):
        pltpu.make_async_copy(m_hbm.at[srcv.at[0]], buf, s_).wait()

    def _scatter(j, buf):
        pltpu.sync_copy(buf, aggsh.at[dstv.at[j]], add=True)

    _gather(0, rows0, sem0)

    @pl.loop(0, CHUNKS_PW // 2 - 1)
    def _pair(k):
        j = 2 * k
        _gather(j + 1, rows1, sem1)
        _drain(rows0, sem0)
        _scatter(j, rows0)
        _gather(j + 2, rows0, sem0)
        _drain(rows1, sem1)
        _scatter(j + 1, rows1)

    _gather(CHUNKS_PW - 1, rows1, sem1)
    _drain(rows0, sem0)
    _scatter(CHUNKS_PW - 2, rows0)
    _drain(rows1, sem1)
    _scatter(CHUNKS_PW - 1, rows1)

    plsc.subcore_barrier()
    pltpu.sync_copy(aggsh.at[pl.ds(s * ROWS_PT, ROWS_PT)],
                    out_hbm.at[c, pl.ds(s * ROWS_PT, ROWS_PT)])


@functools.cache
def _get_sc_segsum():
    return pl.kernel(
        _sc_segsum_body,
        out_type=jax.ShapeDtypeStruct((AGG_PARTS, NPAD, F), jnp.float32),
        mesh=plsc.VectorSubcoreMesh(core_axis_name="c", subcore_axis_name="s",
                                    num_cores=NCORES),
        scratch_types=[
            pltpu.VMEM((CHUNKS_PW, CHUNK), jnp.int32),
            pltpu.VMEM((CHUNKS_PW, CHUNK), jnp.int32),
            pltpu.VMEM((2, CHUNK, F), jnp.float32),
            pltpu.VMEM_SHARED((NPAD, F), jnp.float32),
            pltpu.SemaphoreType.DMA((2,)),
        ],
    )


def _embed_call(x, w0t, conv0):
    return pl.pallas_call(
        _embed_body,
        grid=(GRID,),
        in_specs=[
            pl.BlockSpec((NB, F), lambda i: (i, 0)),
            pl.BlockSpec((F, F), lambda i: (0, 0)),
            pl.BlockSpec((F, F), lambda i: (0, 0)),
        ],
        out_specs=[
            pl.BlockSpec((NB, F), lambda i: (i, 0)),
            pl.BlockSpec((NB, F), lambda i: (i, 0)),
        ],
        out_shape=[
            jax.ShapeDtypeStruct((N, F), jnp.float32),
            jax.ShapeDtypeStruct((N, F), jnp.float32),
        ],
    )(x, w0t, conv0)


def _gru_call(parts, h, wih, whh, bih, bhh, conv, with_conv):
    full = lambda i: (0, 0)
    blk = lambda i: (i, 0)
    out_shape = [jax.ShapeDtypeStruct((N, F), jnp.float32)]
    out_specs = [pl.BlockSpec((NB, F), blk)]
    if with_conv:
        out_shape.append(jax.ShapeDtypeStruct((N, F), jnp.float32))
        out_specs.append(pl.BlockSpec((NB, F), blk))
    return pl.pallas_call(
        functools.partial(_gru_body, with_conv=with_conv),
        grid=(GRID,),
        in_specs=[
            pl.BlockSpec((AGG_PARTS, NB, F), lambda i: (0, i, 0)),
            pl.BlockSpec((NB, F), blk),
            pl.BlockSpec((3 * F, F), full),
            pl.BlockSpec((3 * F, F), full),
            pl.BlockSpec((1, 3 * F), full),
            pl.BlockSpec((1, 3 * F), full),
            pl.BlockSpec((F, F), full),
        ],
        out_specs=out_specs,
        out_shape=out_shape,
    )(parts, h, wih, whh, bih, bhh, conv)


def _head_sample_call(h3, w1, b1, w2, b2, noise):
    return pl.pallas_call(
        _head_sample_body,
        out_shape=jax.ShapeDtypeStruct((G, NPER), jnp.float32),
    )(h3, w1, b1, w2, b2, noise)


def kernel(x, edge_index, batch, num_graphs, W0, conv_weight, gru_w_ih,
           gru_w_hh, gru_b_ih, gru_b_hh, w1, b1, w2, b2):
    bih = gru_b_ih.reshape(1, 3 * F)
    bhh = gru_b_hh.reshape(1, 3 * F)

    src = edge_index[0]
    dst = edge_index[1]
    pad = NCHUNKS_ALLOC * CHUNK - E
    pad_i = jnp.arange(pad, dtype=jnp.int32)
    src_p = jnp.concatenate([src, pad_i % N])
    dst_p = jnp.concatenate([dst, N + pad_i % (NPAD - N)])
    src2 = src_p.reshape(NCHUNKS_ALLOC, CHUNK)
    dst2 = dst_p.reshape(NCHUNKS_ALLOC, CHUNK)
    zeros = jnp.zeros((NPAD, F), jnp.float32)

    h, m = _embed_call(x, W0, conv_weight[0])
    sc_segsum = _get_sc_segsum()
    for i in range(3):
        parts = sc_segsum(m, src2, dst2, zeros)
        with_conv = i < 2
        conv_next = conv_weight[i + 1] if with_conv else conv_weight[0]
        res = _gru_call(parts, h, gru_w_ih, gru_w_hh, bih, bhh,
                        conv_next, with_conv)
        if with_conv:
            h, m = res
        else:
            h = res[0] if isinstance(res, (list, tuple)) else res

    h3 = h.reshape(G, NPER, F)
    noise = jax.random.normal(jax.random.key(42), (G, NPER - 1), jnp.float32)
    noise_p = jnp.pad(noise, ((0, 0), (0, 1)))
    pred = _head_sample_call(h3, w1.reshape(1, F), b1.reshape(1, 1),
                             w2.reshape(1, F), b2.reshape(1, 1), noise_p)
    return pred.reshape(-1)


# speedup vs baseline: 3.5460x; 1.0247x over previous
import functools

import numpy as np

import jax
import jax.numpy as jnp
from jax import lax
from jax.experimental import pallas as pl
from jax.experimental.pallas import tpu as pltpu
from jax.experimental.pallas import tpu_sc as plsc

N = 10000
E = 160000
F = 128
G = 100
NPER = 100
NB = 5000
GRID = N // NB

NCORES = 2
AGG_PARTS = 2
CHUNK = 128
CHUNKS_PW = 40
NCHUNKS_ALLOC = 32 * CHUNKS_PW
EPAD = NCHUNKS_ALLOC * CHUNK
NPAD = 10112
ROWS_PT = NPAD // 16

_PAD_N = EPAD - E
_PAD_SRC2 = np.reshape(np.arange(_PAD_N, dtype=np.int32) % N, (-1, CHUNK))
_PAD_DST2 = np.reshape(N + np.arange(_PAD_N, dtype=np.int32) % (NPAD - N),
                       (-1, CHUNK))
_ZEROS = np.zeros((NPAD, F), np.float32)


def _dot_t(a, b):
    return lax.dot_general(a, b, (((1,), (1,)), ((), ())),
                           preferred_element_type=jnp.float32)


def _embed_body(x_ref, w0_ref, conv0_ref, h_ref, m_ref):
    h = jax.nn.sigmoid(_dot_t(x_ref[...], w0_ref[...]))
    h_ref[...] = h
    m_ref[...] = jnp.dot(h, conv0_ref[...], preferred_element_type=jnp.float32)


def _gru_body(parts_ref, h_ref, wih_ref, whh_ref, bih_ref, bhh_ref,
              conv_ref, h_out_ref, m_ref=None, *, with_conv):
    agg = parts_ref[0] + parts_ref[1]
    h = h_ref[...]
    gi = _dot_t(agg, wih_ref[...]) + bih_ref[...]
    gh = _dot_t(h, whh_ref[...]) + bhh_ref[...]
    r = jax.nn.sigmoid(gi[:, :F] + gh[:, :F])
    z = jax.nn.sigmoid(gi[:, F:2 * F] + gh[:, F:2 * F])
    n = jnp.tanh(gi[:, 2 * F:] + r * gh[:, 2 * F:])
    hn = (1.0 - z) * n + z * h
    if with_conv:
        h_out_ref[...] = hn
        m_ref[...] = jnp.dot(hn, conv_ref[...], preferred_element_type=jnp.float32)
    else:
        h_out_ref[...] = hn.reshape(NB // NPER, NPER, F)


def _softplus(x):
    return jnp.maximum(x, 0.0) + jnp.log1p(jnp.exp(-jnp.abs(x)))


def _head_sample_body(h3_ref, w1_ref, b1_ref, w2_ref, b2_ref, noise_ref, out_ref):
    h3 = jnp.maximum(h3_ref[...], 0.0)
    w1 = w1_ref[...].reshape(1, 1, F)
    w2 = w2_ref[...].reshape(1, 1, F)
    mu = jnp.sum(h3 * w1, axis=2) + b1_ref[0, 0]
    sigma = _softplus(jnp.sum(h3 * w2, axis=2) + b2_ref[0, 0])

    col = lax.broadcasted_iota(jnp.int32, (G, NPER), 1)
    row_t = lax.broadcasted_iota(jnp.int32, (NPER, NPER), 0)
    col_t = lax.broadcasted_iota(jnp.int32, (NPER, NPER), 1)
    main = col < (NPER - 1)
    lastc = col == (NPER - 1)

    s = jnp.where(main, sigma, 0.0)
    mus = jnp.where(main, mu, 0.0)
    sn = jnp.sum(jnp.where(lastc, sigma, 0.0), axis=1, keepdims=True)
    mun = jnp.sum(jnp.where(lastc, mu, 0.0), axis=1, keepdims=True)

    d = s + 1e-6
    sum_s = jnp.sum(s, axis=1, keepdims=True)
    sum_mu = jnp.sum(mus, axis=1, keepdims=True)
    tot = sn + sum_s
    c0 = 1.0 / tot
    c = -mun / sn
    rmean = c * s + mus - c0 * (c * sum_s + sum_mu) * s

    t_rev = (row_t >= col_t).astype(jnp.float32)
    t_ex = (row_t < col_t).astype(jnp.float32)
    rev = jnp.dot(s, t_rev, preferred_element_type=jnp.float32)
    cex = jnp.dot(s / d, t_ex, preferred_element_type=jnp.float32)
    t = 1.0 / (-(sn + rev + 1e-6 * cex))
    ell = jnp.sqrt(d + t * s * s)
    w = t * s / ell

    noise = noise_ref[...]
    wn = w * noise
    cum_wn = jnp.dot(wn, t_ex, preferred_element_type=jnp.float32)
    xr = rmean + ell * noise + s * cum_wn
    xr = jnp.where(main, xr, 0.0)
    last = -jnp.sum(xr, axis=1, keepdims=True)
    out_ref[...] = jnp.where(lastc, jnp.broadcast_to(last, (G, NPER)), xr)


def _sc_segsum_body(m_hbm, src_hbm, dst_hbm, zeros_hbm, out_hbm,
                    srcv, dstv, rowsv, aggsh, sem):
    c = lax.axis_index("c")
    s = lax.axis_index("s")
    wid = s * NCORES + c

    pltpu.sync_copy(zeros_hbm.at[pl.ds(s * ROWS_PT, ROWS_PT)],
                    aggsh.at[pl.ds(s * ROWS_PT, ROWS_PT)])

    pltpu.sync_copy(src_hbm.at[pl.ds(wid * CHUNKS_PW, CHUNKS_PW)], srcv)
    pltpu.sync_copy(dst_hbm.at[pl.ds(wid * CHUNKS_PW, CHUNKS_PW)], dstv)
    plsc.subcore_barrier()

    rows0, rows1 = rowsv.at[0], rowsv.at[1]
    sem0, sem1 = sem.at[0], sem.at[1]

    def _gather(j, buf, s_):
        pltpu.async_copy(m_hbm.at[srcv.at[j]], buf, s_)

    def _drain(buf, s_):
        pltpu.make_async_copy(m_hbm.at[srcv.at[0]], buf, s_).wait()

    def _scatter(j, buf):
        pltpu.sync_copy(buf, aggsh.at[dstv.at[j]], add=True)

    _gather(0, rows0, sem0)

    @pl.loop(0, CHUNKS_PW // 2 - 1)
    def _pair(k):
        j = 2 * k
        _gather(j + 1, rows1, sem1)
        _drain(rows0, sem0)
        _scatter(j, rows0)
        _gather(j + 2, rows0, sem0)
        _drain(rows1, sem1)
        _scatter(j + 1, rows1)

    _gather(CHUNKS_PW - 1, rows1, sem1)
    _drain(rows0, sem0)
    _scatter(CHUNKS_PW - 2, rows0)
    _drain(rows1, sem1)
    _scatter(CHUNKS_PW - 1, rows1)

    plsc.subcore_barrier()
    pltpu.sync_copy(aggsh.at[pl.ds(s * ROWS_PT, ROWS_PT)],
                    out_hbm.at[c, pl.ds(s * ROWS_PT, ROWS_PT)])


@functools.cache
def _get_sc_segsum():
    return pl.kernel(
        _sc_segsum_body,
        out_type=jax.ShapeDtypeStruct((AGG_PARTS, NPAD, F), jnp.float32),
        mesh=plsc.VectorSubcoreMesh(core_axis_name="c", subcore_axis_name="s",
                                    num_cores=NCORES),
        scratch_types=[
            pltpu.VMEM((CHUNKS_PW, CHUNK), jnp.int32),
            pltpu.VMEM((CHUNKS_PW, CHUNK), jnp.int32),
            pltpu.VMEM((2, CHUNK, F), jnp.float32),
            pltpu.VMEM_SHARED((NPAD, F), jnp.float32),
            pltpu.SemaphoreType.DMA((2,)),
        ],
    )


def _embed_call(x, w0t, conv0):
    return pl.pallas_call(
        _embed_body,
        grid=(GRID,),
        in_specs=[
            pl.BlockSpec((NB, F), lambda i: (i, 0)),
            pl.BlockSpec((F, F), lambda i: (0, 0)),
            pl.BlockSpec((F, F), lambda i: (0, 0)),
        ],
        out_specs=[
            pl.BlockSpec((NB, F), lambda i: (i, 0)),
            pl.BlockSpec((NB, F), lambda i: (i, 0)),
        ],
        out_shape=[
            jax.ShapeDtypeStruct((N, F), jnp.float32),
            jax.ShapeDtypeStruct((N, F), jnp.float32),
        ],
    )(x, w0t, conv0)


def _gru_call(parts, h, wih, whh, bih, bhh, conv, with_conv):
    full = lambda i: (0, 0)
    blk = lambda i: (i, 0)
    if with_conv:
        out_shape = [jax.ShapeDtypeStruct((N, F), jnp.float32),
                     jax.ShapeDtypeStruct((N, F), jnp.float32)]
        out_specs = [pl.BlockSpec((NB, F), blk), pl.BlockSpec((NB, F), blk)]
    else:
        out_shape = [jax.ShapeDtypeStruct((G, NPER, F), jnp.float32)]
        out_specs = [pl.BlockSpec((NB // NPER, NPER, F),
                                  lambda i: (i, 0, 0))]
    return pl.pallas_call(
        functools.partial(_gru_body, with_conv=with_conv),
        grid=(GRID,),
        in_specs=[
            pl.BlockSpec((AGG_PARTS, NB, F), lambda i: (0, i, 0)),
            pl.BlockSpec((NB, F), blk),
            pl.BlockSpec((3 * F, F), full),
            pl.BlockSpec((3 * F, F), full),
            pl.BlockSpec((1, 3 * F), full),
            pl.BlockSpec((1, 3 * F), full),
            pl.BlockSpec((F, F), full),
        ],
        out_specs=out_specs,
        out_shape=out_shape,
    )(parts, h, wih, whh, bih, bhh, conv)


def _head_sample_call(h3, w1, b1, w2, b2, noise):
    return pl.pallas_call(
        _head_sample_body,
        out_shape=jax.ShapeDtypeStruct((G, NPER), jnp.float32),
    )(h3, w1, b1, w2, b2, noise)


def kernel(x, edge_index, batch, num_graphs, W0, conv_weight, gru_w_ih,
           gru_w_hh, gru_b_ih, gru_b_hh, w1, b1, w2, b2):
    bih = gru_b_ih.reshape(1, 3 * F)
    bhh = gru_b_hh.reshape(1, 3 * F)

    src2 = jnp.concatenate([edge_index[0].reshape(E // CHUNK, CHUNK),
                            jnp.asarray(_PAD_SRC2)])
    dst2 = jnp.concatenate([edge_index[1].reshape(E // CHUNK, CHUNK),
                            jnp.asarray(_PAD_DST2)])
    zeros = jnp.asarray(_ZEROS)

    h, m = _embed_call(x, W0, conv_weight[0])
    sc_segsum = _get_sc_segsum()
    for i in range(3):
        parts = sc_segsum(m, src2, dst2, zeros)
        with_conv = i < 2
        conv_next = conv_weight[i + 1] if with_conv else conv_weight[0]
        res = _gru_call(parts, h, gru_w_ih, gru_w_hh, bih, bhh,
                        conv_next, with_conv)
        if with_conv:
            h, m = res
        else:
            h3 = res[0] if isinstance(res, (list, tuple)) else res

    noise = jax.random.normal(jax.random.key(42), (G, NPER - 1), jnp.float32)
    noise_p = jnp.pad(noise, ((0, 0), (0, 1)))
    pred = _head_sample_call(h3, w1.reshape(1, F), b1.reshape(1, 1),
                             w2.reshape(1, F), b2.reshape(1, 1), noise_p)
    return pred.reshape(-1)
